# Initial kernel scaffold; baseline (speedup 1.0000x reference)
import jax
import jax.numpy as jnp
from jax.experimental import pallas as pl


def kernel(pos, batch, params):
    raise NotImplementedError("write your pallas kernel here")


# speedup vs baseline: 10.3456x; 10.3456x over previous
import functools

import jax
import jax.numpy as jnp
from jax.experimental import pallas as pl
from jax.experimental.pallas import tpu as pltpu

B = 16
P = 1024
S1 = 512
S2 = 128
K = 64
R1SQ = 0.2 * 0.2
R2SQ = 0.4 * 0.4
CBLK = 128

_f32 = jnp.float32


def _fps_plane(px, py, pz, S):
    Bc, N = px.shape
    iota = jax.lax.broadcasted_iota(jnp.int32, (Bc, N), 1)
    iota_s = jax.lax.broadcasted_iota(jnp.int32, (Bc, S), 1)
    lx, ly, lz = px[:, 0:1], py[:, 0:1], pz[:, 0:1]
    cx = jnp.where(iota_s == 0, lx, 0.0)
    cy = jnp.where(iota_s == 0, ly, 0.0)
    cz = jnp.where(iota_s == 0, lz, 0.0)
    mind = jnp.full((Bc, N), jnp.inf, _f32)

    def body(i, st):
        mind, lx, ly, lz, cx, cy, cz = st
        d = (px - lx) ** 2 + (py - ly) ** 2 + (pz - lz) ** 2
        mind = jnp.minimum(mind, d)
        m = jnp.max(mind, axis=1, keepdims=True)
        idx = jnp.min(jnp.where(mind == m, iota, N), axis=1, keepdims=True)
        oh = iota == idx
        lx = jnp.sum(jnp.where(oh, px, 0.0), axis=1, keepdims=True)
        ly = jnp.sum(jnp.where(oh, py, 0.0), axis=1, keepdims=True)
        lz = jnp.sum(jnp.where(oh, pz, 0.0), axis=1, keepdims=True)
        cx = jnp.where(iota_s == i, lx, cx)
        cy = jnp.where(iota_s == i, ly, cy)
        cz = jnp.where(iota_s == i, lz, cz)
        return (mind, lx, ly, lz, cx, cy, cz)

    st = jax.lax.fori_loop(1, S, body, (mind, lx, ly, lz, cx, cy, cz))
    return st[4], st[5], st[6]


def _fps_kernel(px_ref, py_ref, pz_ref,
                c1x_ref, c1y_ref, c1z_ref,
                c2x_ref, c2y_ref, c2z_ref):
    c1x, c1y, c1z = _fps_plane(px_ref[:, :], py_ref[:, :], pz_ref[:, :], S1)
    c1x_ref[:, :] = c1x
    c1y_ref[:, :] = c1y
    c1z_ref[:, :] = c1z
    c2x, c2y, c2z = _fps_plane(c1x, c1y, c1z, S2)
    c2x_ref[:, :] = c2x
    c2y_ref[:, :] = c2y
    c2z_ref[:, :] = c2z


def _fold_max(msg, nslab):
    n = nslab
    while n > 1:
        half = n // 2
        msg = jnp.maximum(msg[:half], msg[half:n])
        n = half
    return msg.reshape(msg.shape[1], msg.shape[2])


def _sa1_kernel(px_ref, py_ref, pz_ref, pm_ref,
                cx_ref, cy_ref, cz_ref,
                w1_ref, b1_ref, w2_ref, b2_ref, w3_ref, b3_ref,
                out_ref, featp_ref):
    prow_x = px_ref[0]
    prow_y = py_ref[0]
    prow_z = pz_ref[0]
    cxc = cx_ref[0, 0]
    cyc = cy_ref[0, 0]
    czc = cz_ref[0, 0]
    d2 = (cxc - prow_x) ** 2 + (cyc - prow_y) ** 2 + (czc - prow_z) ** 2
    d2m = jnp.where(d2 <= R1SQ, d2, jnp.inf)
    iota = jax.lax.broadcasted_iota(jnp.int32, (CBLK, P), 1)
    pm = pm_ref[0]
    ccat = jnp.concatenate([cxc, cyc, czc], axis=1)
    for k in range(K):
        m = jnp.min(d2m, axis=1, keepdims=True)
        idx = jnp.min(jnp.where(d2m == m, iota, P), axis=1, keepdims=True)
        oh = iota == idx
        g = jnp.dot(oh.astype(_f32), pm, preferred_element_type=_f32)
        rel = jnp.where(m <= R1SQ, g - ccat, 0.0)
        featp_ref[k * CBLK:(k + 1) * CBLK, :] = rel
        d2m = jnp.where(oh, jnp.inf, d2m)
    feat = featp_ref[:, :]
    h = jnp.maximum(jnp.dot(feat, w1_ref[:, :], preferred_element_type=_f32)
                    + b1_ref[:, :], 0.0)
    h = jnp.maximum(jnp.dot(h, w2_ref[:, :], preferred_element_type=_f32)
                    + b2_ref[:, :], 0.0)
    h = jnp.dot(h, w3_ref[:, :], preferred_element_type=_f32) + b3_ref[:, :]
    out_ref[0, 0] = _fold_max(h.reshape(K, CBLK, 128), K)


def _sa2_kernel(c1x_ref, c1y_ref, c1z_ref, c1m_ref, x1_ref,
                cx_ref, cy_ref, cz_ref,
                w1a_ref, w1b_ref, b1_ref, w2_ref, b2_ref, w3_ref, b3_ref,
                out_ref, featx_ref, featp_ref):
    prow_x = c1x_ref[0]
    prow_y = c1y_ref[0]
    prow_z = c1z_ref[0]
    cxc = cx_ref[0]
    cyc = cy_ref[0]
    czc = cz_ref[0]
    d2 = (cxc - prow_x) ** 2 + (cyc - prow_y) ** 2 + (czc - prow_z) ** 2
    d2m = jnp.where(d2 <= R2SQ, d2, jnp.inf)
    iota = jax.lax.broadcasted_iota(jnp.int32, (S2, S1), 1)
    c1m = c1m_ref[0]
    x1 = x1_ref[0]
    ccat = jnp.concatenate([cxc, cyc, czc], axis=1)
    gx0 = None
    for k in range(K):
        m = jnp.min(d2m, axis=1, keepdims=True)
        idx = jnp.min(jnp.where(d2m == m, iota, S1), axis=1, keepdims=True)
        oh = (iota == idx).astype(_f32)
        gx = jnp.dot(oh, x1, preferred_element_type=_f32)
        gp = jnp.dot(oh, c1m, preferred_element_type=_f32) - ccat
        valid = m <= R2SQ
        if k == 0:
            gx0 = gx
        else:
            gx = jnp.where(valid, gx, gx0)
        featx_ref[k * S2:(k + 1) * S2, :] = gx
        featp_ref[k * S2:(k + 1) * S2, :] = jnp.where(valid, gp, 0.0)
        d2m = jnp.where(iota == idx, jnp.inf, d2m)
    h = jnp.maximum(
        jnp.dot(featx_ref[:, :], w1a_ref[:, :], preferred_element_type=_f32)
        + jnp.dot(featp_ref[:, :], w1b_ref[:, :], preferred_element_type=_f32)
        + b1_ref[:, :], 0.0)
    h = jnp.maximum(jnp.dot(h, w2_ref[:, :], preferred_element_type=_f32)
                    + b2_ref[:, :], 0.0)
    h = jnp.dot(h, w3_ref[:, :], preferred_element_type=_f32) + b3_ref[:, :]
    out_ref[0] = _fold_max(h.reshape(K, S2, 256), K)


def _head_kernel(x2_ref, c2m_ref,
                 wa_ref, wb_ref, b1_ref, w2_ref, b2_ref, w3_ref, b3_ref,
                 h1_ref, hb1_ref, h2_ref, hb2_ref, h3_ref, hb3_ref,
                 out_ref):
    x2 = x2_ref[:, :, :].reshape(B * S2, 256)
    c2 = c2m_ref[:, :, :].reshape(B * S2, 3)
    h = jnp.maximum(jnp.dot(x2, wa_ref[:, :], preferred_element_type=_f32)
                    + jnp.dot(c2, wb_ref[:, :], preferred_element_type=_f32)
                    + b1_ref[:, :], 0.0)
    h = jnp.maximum(jnp.dot(h, w2_ref[:, :], preferred_element_type=_f32)
                    + b2_ref[:, :], 0.0)
    h = jnp.dot(h, w3_ref[:, :], preferred_element_type=_f32) + b3_ref[:, :]
    pooled = h.reshape(B, S2, 1024)
    n = S2
    while n > 1:
        half = n // 2
        pooled = jnp.maximum(pooled[:, :half], pooled[:, half:n])
        n = half
    pooled = pooled.reshape(B, 1024)
    o = jnp.maximum(jnp.dot(pooled, h1_ref[:, :], preferred_element_type=_f32)
                    + hb1_ref[:, :], 0.0)
    o = jnp.maximum(jnp.dot(o, h2_ref[:, :], preferred_element_type=_f32)
                    + hb2_ref[:, :], 0.0)
    out_ref[:, :] = (jnp.dot(o, h3_ref[:, :], preferred_element_type=_f32)
                     + hb3_ref[:, :])


def _b2(b):
    return b.reshape(1, -1)


@jax.jit
def kernel(pos, batch, params):
    del batch
    pos3 = pos.reshape(B, P, 3)
    px = pos3[:, :, 0]
    py = pos3[:, :, 1]
    pz = pos3[:, :, 2]

    fps = pl.pallas_call(
        _fps_kernel,
        out_shape=(
            jax.ShapeDtypeStruct((B, S1), _f32),
            jax.ShapeDtypeStruct((B, S1), _f32),
            jax.ShapeDtypeStruct((B, S1), _f32),
            jax.ShapeDtypeStruct((B, S2), _f32),
            jax.ShapeDtypeStruct((B, S2), _f32),
            jax.ShapeDtypeStruct((B, S2), _f32),
        ),
    )
    c1x, c1y, c1z, c2x, c2y, c2z = fps(px, py, pz)

    nblk = S1 // CBLK
    c1x_b = c1x.reshape(B, nblk, CBLK, 1)
    c1y_b = c1y.reshape(B, nblk, CBLK, 1)
    c1z_b = c1z.reshape(B, nblk, CBLK, 1)
    w1, b1 = params['sa1'][0]
    w2, b2 = params['sa1'][1]
    w3, b3 = params['sa1'][2]

    x1 = pl.pallas_call(
        _sa1_kernel,
        grid=(B, nblk),
        in_specs=[
            pl.BlockSpec((1, 1, P), lambda b, j: (b, 0, 0)),
            pl.BlockSpec((1, 1, P), lambda b, j: (b, 0, 0)),
            pl.BlockSpec((1, 1, P), lambda b, j: (b, 0, 0)),
            pl.BlockSpec((1, P, 3), lambda b, j: (b, 0, 0)),
            pl.BlockSpec((1, 1, CBLK, 1), lambda b, j: (b, j, 0, 0)),
            pl.BlockSpec((1, 1, CBLK, 1), lambda b, j: (b, j, 0, 0)),
            pl.BlockSpec((1, 1, CBLK, 1), lambda b, j: (b, j, 0, 0)),
            pl.BlockSpec((3, 64), lambda b, j: (0, 0)),
            pl.BlockSpec((1, 64), lambda b, j: (0, 0)),
            pl.BlockSpec((64, 64), lambda b, j: (0, 0)),
            pl.BlockSpec((1, 64), lambda b, j: (0, 0)),
            pl.BlockSpec((64, 128), lambda b, j: (0, 0)),
            pl.BlockSpec((1, 128), lambda b, j: (0, 0)),
        ],
        out_specs=pl.BlockSpec((1, 1, CBLK, 128), lambda b, j: (b, j, 0, 0)),
        out_shape=jax.ShapeDtypeStruct((B, nblk, CBLK, 128), _f32),
        scratch_shapes=[pltpu.VMEM((K * CBLK, 3), _f32)],
    )(px.reshape(B, 1, P), py.reshape(B, 1, P), pz.reshape(B, 1, P),
      pos3, c1x_b, c1y_b, c1z_b,
      w1, _b2(b1), w2, _b2(b2), w3, _b2(b3))

    x1f = x1.reshape(B, S1, 128)
    c1m = jnp.stack([c1x, c1y, c1z], axis=-1)
    c2x_b = c2x.reshape(B, S2, 1)
    c2y_b = c2y.reshape(B, S2, 1)
    c2z_b = c2z.reshape(B, S2, 1)
    w1_2, b1_2 = params['sa2'][0]
    w2_2, b2_2 = params['sa2'][1]
    w3_2, b3_2 = params['sa2'][2]

    x2 = pl.pallas_call(
        _sa2_kernel,
        grid=(B,),
        in_specs=[
            pl.BlockSpec((1, 1, S1), lambda b: (b, 0, 0)),
            pl.BlockSpec((1, 1, S1), lambda b: (b, 0, 0)),
            pl.BlockSpec((1, 1, S1), lambda b: (b, 0, 0)),
            pl.BlockSpec((1, S1, 3), lambda b: (b, 0, 0)),
            pl.BlockSpec((1, S1, 128), lambda b: (b, 0, 0)),
            pl.BlockSpec((1, S2, 1), lambda b: (b, 0, 0)),
            pl.BlockSpec((1, S2, 1), lambda b: (b, 0, 0)),
            pl.BlockSpec((1, S2, 1), lambda b: (b, 0, 0)),
            pl.BlockSpec((128, 128), lambda b: (0, 0)),
            pl.BlockSpec((3, 128), lambda b: (0, 0)),
            pl.BlockSpec((1, 128), lambda b: (0, 0)),
            pl.BlockSpec((128, 128), lambda b: (0, 0)),
            pl.BlockSpec((1, 128), lambda b: (0, 0)),
            pl.BlockSpec((128, 256), lambda b: (0, 0)),
            pl.BlockSpec((1, 256), lambda b: (0, 0)),
        ],
        out_specs=pl.BlockSpec((1, S2, 256), lambda b: (b, 0, 0)),
        out_shape=jax.ShapeDtypeStruct((B, S2, 256), _f32),
        scratch_shapes=[pltpu.VMEM((K * S2, 128), _f32),
                        pltpu.VMEM((K * S2, 3), _f32)],
    )(c1x.reshape(B, 1, S1), c1y.reshape(B, 1, S1), c1z.reshape(B, 1, S1),
      c1m, x1f, c2x_b, c2y_b, c2z_b,
      w1_2[:128], w1_2[128:], _b2(b1_2), w2_2, _b2(b2_2), w3_2, _b2(b3_2))

    c2m = jnp.stack([c2x, c2y, c2z], axis=-1)
    wa, ba = params['sa3'][0]
    w2_3, b2_3 = params['sa3'][1]
    w3_3, b3_3 = params['sa3'][2]
    h1, hb1 = params['head'][0]
    h2, hb2 = params['head'][1]
    h3, hb3 = params['head'][2]

    out = pl.pallas_call(
        _head_kernel,
        out_shape=jax.ShapeDtypeStruct((B, 32), _f32),
    )(x2, c2m, wa[:256], wa[256:], _b2(ba), w2_3, _b2(b2_3), w3_3, _b2(b3_3),
      h1, _b2(hb1), h2, _b2(hb2), h3, _b2(hb3))
    return out


# speedup vs baseline: 16.5425x; 1.5990x over previous
import functools

import jax
import jax.numpy as jnp
from jax.experimental import pallas as pl
from jax.experimental.pallas import tpu as pltpu

B = 16
P = 1024
S1 = 512
S2 = 128
K = 64
R1SQ = 0.2 * 0.2
R2SQ = 0.4 * 0.4
CBLK = 128

_f32 = jnp.float32


def _fps_plane(px, py, pz, S):
    Bc, N = px.shape
    iotaf = jax.lax.broadcasted_iota(jnp.int32, (Bc, N), 1).astype(_f32)
    iota_s = jax.lax.broadcasted_iota(jnp.int32, (Bc, S), 1).astype(_f32)
    lx, ly, lz = px[:, 0:1], py[:, 0:1], pz[:, 0:1]
    cx = jnp.where(iota_s == 0, lx, 0.0)
    cy = jnp.where(iota_s == 0, ly, 0.0)
    cz = jnp.where(iota_s == 0, lz, 0.0)
    mind = jnp.full((Bc, N), jnp.inf, _f32)

    def body(i, st):
        mind, lx, ly, lz, cx, cy, cz = st
        d = (px - lx) ** 2 + (py - ly) ** 2 + (pz - lz) ** 2
        mind = jnp.minimum(mind, d)
        m = jnp.max(mind, axis=1, keepdims=True)
        tie = jnp.where(mind == m, iotaf, float(N))
        idxf = jnp.min(tie, axis=1, keepdims=True)
        oh = tie == idxf
        lx = jnp.sum(jnp.where(oh, px, 0.0), axis=1, keepdims=True)
        ly = jnp.sum(jnp.where(oh, py, 0.0), axis=1, keepdims=True)
        lz = jnp.sum(jnp.where(oh, pz, 0.0), axis=1, keepdims=True)
        i_f = i.astype(_f32)
        cx = jnp.where(iota_s == i_f, lx, cx)
        cy = jnp.where(iota_s == i_f, ly, cy)
        cz = jnp.where(iota_s == i_f, lz, cz)
        return (mind, lx, ly, lz, cx, cy, cz)

    st = jax.lax.fori_loop(1, S, body, (mind, lx, ly, lz, cx, cy, cz))
    return st[4], st[5], st[6]


def _fps_kernel(px_ref, py_ref, pz_ref,
                c1x_ref, c1y_ref, c1z_ref,
                c2x_ref, c2y_ref, c2z_ref):
    c1x, c1y, c1z = _fps_plane(px_ref[:, :], py_ref[:, :], pz_ref[:, :], S1)
    c1x_ref[:, :] = c1x
    c1y_ref[:, :] = c1y
    c1z_ref[:, :] = c1z
    c2x, c2y, c2z = _fps_plane(c1x, c1y, c1z, S2)
    c2x_ref[:, :] = c2x
    c2y_ref[:, :] = c2y
    c2z_ref[:, :] = c2z


def _lane_cumsum(x, n):
    sh = 1
    while sh < n:
        shifted = jnp.concatenate(
            [jnp.zeros(x.shape[:-1] + (sh,), x.dtype), x[..., :n - sh]],
            axis=-1)
        x = x + shifted
        sh *= 2
    return x


def _topk_slots(d2m, kk):
    R, N = d2m.shape
    bits = jax.lax.bitcast_convert_type(d2m, jnp.int32)
    T = jnp.zeros((R, 1), jnp.int32)
    for b in range(30, -1, -1):
        cand = T | (1 << b)
        cnt = jnp.sum((bits < cand).astype(_f32), axis=1, keepdims=True)
        T = jnp.where(cnt < kk, cand, T)
    ltb = bits < T
    cnt_lt = jnp.sum(ltb.astype(_f32), axis=1, keepdims=True)
    need = kk - cnt_lt
    tieb = bits == T
    tier = _lane_cumsum(tieb.astype(_f32), N)
    maskb = (ltb | (tieb & (tier <= need))) & (d2m < jnp.inf)
    maskf = maskb.astype(_f32)
    slot = _lane_cumsum(maskf, N)
    slotm = jnp.where(maskb, slot, 0.0)
    cntv = slot[:, N - 1:N]
    return slotm, cntv


def _fold_max(msg, nslab):
    n = nslab
    while n > 1:
        half = n // 2
        msg = jnp.maximum(msg[:half], msg[half:n])
        n = half
    return msg.reshape(msg.shape[1], msg.shape[2])


def _sa1_kernel(px_ref, py_ref, pz_ref, pm_ref,
                cx_ref, cy_ref, cz_ref,
                w1_ref, b1_ref, w2_ref, b2_ref, w3_ref, b3_ref,
                out_ref, featp_ref):
    prow_x = px_ref[0]
    prow_y = py_ref[0]
    prow_z = pz_ref[0]
    cxc = cx_ref[0, 0]
    cyc = cy_ref[0, 0]
    czc = cz_ref[0, 0]
    d2 = (cxc - prow_x) ** 2 + (cyc - prow_y) ** 2 + (czc - prow_z) ** 2
    d2m = jnp.where(d2 <= R1SQ, d2, jnp.inf)
    pm = pm_ref[0]
    ccat = jnp.concatenate([cxc, cyc, czc], axis=1)
    slotm, cntv = _topk_slots(d2m, K)
    rel0 = None
    for k in range(K):
        ohf = (slotm == float(k + 1)).astype(_f32)
        g = jnp.dot(ohf, pm, preferred_element_type=_f32)
        rel = g - ccat
        if k == 0:
            rel0 = rel
        else:
            rel = jnp.where(cntv > float(k), rel, rel0)
        featp_ref[k * CBLK:(k + 1) * CBLK, :] = rel
    feat = featp_ref[:, :]
    h = jnp.maximum(jnp.dot(feat, w1_ref[:, :], preferred_element_type=_f32)
                    + b1_ref[:, :], 0.0)
    h = jnp.maximum(jnp.dot(h, w2_ref[:, :], preferred_element_type=_f32)
                    + b2_ref[:, :], 0.0)
    h = jnp.dot(h, w3_ref[:, :], preferred_element_type=_f32) + b3_ref[:, :]
    out_ref[0, 0] = _fold_max(h.reshape(K, CBLK, 128), K)


def _sa2_kernel(c1x_ref, c1y_ref, c1z_ref, c1m_ref, x1_ref,
                cx_ref, cy_ref, cz_ref,
                w1a_ref, w1b_ref, b1_ref, w2_ref, b2_ref, w3_ref, b3_ref,
                out_ref, featx_ref, featp_ref):
    prow_x = c1x_ref[0]
    prow_y = c1y_ref[0]
    prow_z = c1z_ref[0]
    cxc = cx_ref[0]
    cyc = cy_ref[0]
    czc = cz_ref[0]
    d2 = (cxc - prow_x) ** 2 + (cyc - prow_y) ** 2 + (czc - prow_z) ** 2
    d2m = jnp.where(d2 <= R2SQ, d2, jnp.inf)
    c1m = c1m_ref[0]
    x1 = x1_ref[0]
    ccat = jnp.concatenate([cxc, cyc, czc], axis=1)
    slotm, cntv = _topk_slots(d2m, K)
    gx0 = None
    gp0 = None
    for k in range(K):
        ohf = (slotm == float(k + 1)).astype(_f32)
        gx = jnp.dot(ohf, x1, preferred_element_type=_f32)
        gp = jnp.dot(ohf, c1m, preferred_element_type=_f32) - ccat
        if k == 0:
            gx0, gp0 = gx, gp
        else:
            fill = cntv > float(k)
            gx = jnp.where(fill, gx, gx0)
            gp = jnp.where(fill, gp, gp0)
        featx_ref[k * S2:(k + 1) * S2, :] = gx
        featp_ref[k * S2:(k + 1) * S2, :] = gp
    h = jnp.maximum(
        jnp.dot(featx_ref[:, :], w1a_ref[:, :], preferred_element_type=_f32)
        + jnp.dot(featp_ref[:, :], w1b_ref[:, :], preferred_element_type=_f32)
        + b1_ref[:, :], 0.0)
    h = jnp.maximum(jnp.dot(h, w2_ref[:, :], preferred_element_type=_f32)
                    + b2_ref[:, :], 0.0)
    h = jnp.dot(h, w3_ref[:, :], preferred_element_type=_f32) + b3_ref[:, :]
    out_ref[0] = _fold_max(h.reshape(K, S2, 256), K)


def _head_kernel(x2_ref, c2m_ref,
                 wa_ref, wb_ref, b1_ref, w2_ref, b2_ref, w3_ref, b3_ref,
                 h1_ref, hb1_ref, h2_ref, hb2_ref, h3_ref, hb3_ref,
                 out_ref):
    x2 = x2_ref[:, :, :].reshape(B * S2, 256)
    c2 = c2m_ref[:, :, :].reshape(B * S2, 3)
    h = jnp.maximum(jnp.dot(x2, wa_ref[:, :], preferred_element_type=_f32)
                    + jnp.dot(c2, wb_ref[:, :], preferred_element_type=_f32)
                    + b1_ref[:, :], 0.0)
    h = jnp.maximum(jnp.dot(h, w2_ref[:, :], preferred_element_type=_f32)
                    + b2_ref[:, :], 0.0)
    h = jnp.dot(h, w3_ref[:, :], preferred_element_type=_f32) + b3_ref[:, :]
    pooled = h.reshape(B, S2, 1024)
    n = S2
    while n > 1:
        half = n // 2
        pooled = jnp.maximum(pooled[:, :half], pooled[:, half:n])
        n = half
    pooled = pooled.reshape(B, 1024)
    o = jnp.maximum(jnp.dot(pooled, h1_ref[:, :], preferred_element_type=_f32)
                    + hb1_ref[:, :], 0.0)
    o = jnp.maximum(jnp.dot(o, h2_ref[:, :], preferred_element_type=_f32)
                    + hb2_ref[:, :], 0.0)
    out_ref[:, :] = (jnp.dot(o, h3_ref[:, :], preferred_element_type=_f32)
                     + hb3_ref[:, :])


def _b2(b):
    return b.reshape(1, -1)


@jax.jit
def kernel(pos, batch, params):
    del batch
    pos3 = pos.reshape(B, P, 3)
    px = pos3[:, :, 0]
    py = pos3[:, :, 1]
    pz = pos3[:, :, 2]

    fps = pl.pallas_call(
        _fps_kernel,
        out_shape=(
            jax.ShapeDtypeStruct((B, S1), _f32),
            jax.ShapeDtypeStruct((B, S1), _f32),
            jax.ShapeDtypeStruct((B, S1), _f32),
            jax.ShapeDtypeStruct((B, S2), _f32),
            jax.ShapeDtypeStruct((B, S2), _f32),
            jax.ShapeDtypeStruct((B, S2), _f32),
        ),
    )
    c1x, c1y, c1z, c2x, c2y, c2z = fps(px, py, pz)

    nblk = S1 // CBLK
    c1x_b = c1x.reshape(B, nblk, CBLK, 1)
    c1y_b = c1y.reshape(B, nblk, CBLK, 1)
    c1z_b = c1z.reshape(B, nblk, CBLK, 1)
    w1, b1 = params['sa1'][0]
    w2, b2 = params['sa1'][1]
    w3, b3 = params['sa1'][2]

    x1 = pl.pallas_call(
        _sa1_kernel,
        grid=(B, nblk),
        in_specs=[
            pl.BlockSpec((1, 1, P), lambda b, j: (b, 0, 0)),
            pl.BlockSpec((1, 1, P), lambda b, j: (b, 0, 0)),
            pl.BlockSpec((1, 1, P), lambda b, j: (b, 0, 0)),
            pl.BlockSpec((1, P, 3), lambda b, j: (b, 0, 0)),
            pl.BlockSpec((1, 1, CBLK, 1), lambda b, j: (b, j, 0, 0)),
            pl.BlockSpec((1, 1, CBLK, 1), lambda b, j: (b, j, 0, 0)),
            pl.BlockSpec((1, 1, CBLK, 1), lambda b, j: (b, j, 0, 0)),
            pl.BlockSpec((3, 64), lambda b, j: (0, 0)),
            pl.BlockSpec((1, 64), lambda b, j: (0, 0)),
            pl.BlockSpec((64, 64), lambda b, j: (0, 0)),
            pl.BlockSpec((1, 64), lambda b, j: (0, 0)),
            pl.BlockSpec((64, 128), lambda b, j: (0, 0)),
            pl.BlockSpec((1, 128), lambda b, j: (0, 0)),
        ],
        out_specs=pl.BlockSpec((1, 1, CBLK, 128), lambda b, j: (b, j, 0, 0)),
        out_shape=jax.ShapeDtypeStruct((B, nblk, CBLK, 128), _f32),
        scratch_shapes=[pltpu.VMEM((K * CBLK, 3), _f32)],
    )(px.reshape(B, 1, P), py.reshape(B, 1, P), pz.reshape(B, 1, P),
      pos3, c1x_b, c1y_b, c1z_b,
      w1, _b2(b1), w2, _b2(b2), w3, _b2(b3))

    x1f = x1.reshape(B, S1, 128)
    c1m = jnp.stack([c1x, c1y, c1z], axis=-1)
    c2x_b = c2x.reshape(B, S2, 1)
    c2y_b = c2y.reshape(B, S2, 1)
    c2z_b = c2z.reshape(B, S2, 1)
    w1_2, b1_2 = params['sa2'][0]
    w2_2, b2_2 = params['sa2'][1]
    w3_2, b3_2 = params['sa2'][2]

    x2 = pl.pallas_call(
        _sa2_kernel,
        grid=(B,),
        in_specs=[
            pl.BlockSpec((1, 1, S1), lambda b: (b, 0, 0)),
            pl.BlockSpec((1, 1, S1), lambda b: (b, 0, 0)),
            pl.BlockSpec((1, 1, S1), lambda b: (b, 0, 0)),
            pl.BlockSpec((1, S1, 3), lambda b: (b, 0, 0)),
            pl.BlockSpec((1, S1, 128), lambda b: (b, 0, 0)),
            pl.BlockSpec((1, S2, 1), lambda b: (b, 0, 0)),
            pl.BlockSpec((1, S2, 1), lambda b: (b, 0, 0)),
            pl.BlockSpec((1, S2, 1), lambda b: (b, 0, 0)),
            pl.BlockSpec((128, 128), lambda b: (0, 0)),
            pl.BlockSpec((3, 128), lambda b: (0, 0)),
            pl.BlockSpec((1, 128), lambda b: (0, 0)),
            pl.BlockSpec((128, 128), lambda b: (0, 0)),
            pl.BlockSpec((1, 128), lambda b: (0, 0)),
            pl.BlockSpec((128, 256), lambda b: (0, 0)),
            pl.BlockSpec((1, 256), lambda b: (0, 0)),
        ],
        out_specs=pl.BlockSpec((1, S2, 256), lambda b: (b, 0, 0)),
        out_shape=jax.ShapeDtypeStruct((B, S2, 256), _f32),
        scratch_shapes=[pltpu.VMEM((K * S2, 128), _f32),
                        pltpu.VMEM((K * S2, 3), _f32)],
    )(c1x.reshape(B, 1, S1), c1y.reshape(B, 1, S1), c1z.reshape(B, 1, S1),
      c1m, x1f, c2x_b, c2y_b, c2z_b,
      w1_2[:128], w1_2[128:], _b2(b1_2), w2_2, _b2(b2_2), w3_2, _b2(b3_2))

    c2m = jnp.stack([c2x, c2y, c2z], axis=-1)
    wa, ba = params['sa3'][0]
    w2_3, b2_3 = params['sa3'][1]
    w3_3, b3_3 = params['sa3'][2]
    h1, hb1 = params['head'][0]
    h2, hb2 = params['head'][1]
    h3, hb3 = params['head'][2]

    out = pl.pallas_call(
        _head_kernel,
        out_shape=jax.ShapeDtypeStruct((B, 32), _f32),
    )(x2, c2m, wa[:256], wa[256:], _b2(ba), w2_3, _b2(b2_3), w3_3, _b2(b3_3),
      h1, _b2(hb1), h2, _b2(hb2), h3, _b2(hb3))
    return out
